# baseline (speedup 1.0000x reference)
import functools

import jax
import jax.numpy as jnp
from jax import lax
from jax.experimental import pallas as pl
from jax.experimental.pallas import tpu as pltpu
from jax.experimental.pallas import tpu_sc as plsc

DIM = 64
BATCH = 16384
NENT = 1000000

_KH = 8192
_NBLK = -(-NENT // (2 * _KH))
SPLITP = (_NBLK - 1) * _KH
_NROWS = _NBLK * _KH

_INFO = plsc.get_sparse_core_info()
_NC = _INFO.num_cores
_NS = _INFO.num_subcores
_NW = _NC * _NS
_PER_W = BATCH // _NW
_CHUNK = 256
_NCH = _PER_W // _CHUNK
_EPS = 1e-12


def _detile_body(x1_ref, x2_ref, o_ref):
    eye = jnp.eye(DIM, dtype=jnp.float32)
    dims = (((0,), (0,)), ((), ()))
    y1 = jax.lax.dot_general(x1_ref[...], eye, dims,
                             preferred_element_type=jnp.float32)
    y2 = jax.lax.dot_general(x2_ref[...], eye, dims,
                             preferred_element_type=jnp.float32)
    o_ref[...] = jnp.concatenate([y1, y2], axis=1)


def _tc_detile(entT):
    return pl.pallas_call(
        _detile_body,
        grid=(_NBLK,),
        in_specs=[pl.BlockSpec((DIM, _KH), lambda i: (0, i)),
                  pl.BlockSpec((DIM, _KH), lambda i: (0, i + _NBLK - 1))],
        out_specs=pl.BlockSpec((_KH, 2 * DIM), lambda i: (i, 0)),
        out_shape=jax.ShapeDtypeStruct((_NROWS, 2 * DIM), jnp.float32),
    )(entT, entT)


def _gather_body(hidx_hbm, ridx_hbm, tidx_hbm, ent_hbm, rel_hbm,
                 hrows_hbm, rrows_hbm, trows_hbm,
                 idxh_v, idxr_v, idxt_v, bufh_v, bufr_v, buft_v,
                 semh, semr, semt):
    wid = lax.axis_index("s") * _NC + lax.axis_index("c")
    base = wid * _PER_W

    pltpu.sync_copy(hidx_hbm.at[pl.ds(base, _PER_W)], idxh_v)
    pltpu.sync_copy(ridx_hbm.at[pl.ds(base, _PER_W)], idxr_v)
    pltpu.sync_copy(tidx_hbm.at[pl.ds(base, _PER_W)], idxt_v)

    def make_fire(idx_v, tab_hbm, buf_v, sem, off):
        def fire(g, carry):
            v = idx_v[pl.ds(off + g * 16, 16)]
            for i in range(16):
                pltpu.async_copy(tab_hbm.at[pl.ds(v[i], 1)],
                                 buf_v.at[pl.ds(g * 16 + i, 1)], sem)
            return carry
        return fire

    for j in range(_NCH):
        off = j * _CHUNK
        lax.fori_loop(0, _CHUNK // 16, make_fire(idxh_v, ent_hbm, bufh_v, semh, off), 0)
        lax.fori_loop(0, _CHUNK // 16, make_fire(idxt_v, ent_hbm, buft_v, semt, off), 0)
        lax.fori_loop(0, _CHUNK // 16, make_fire(idxr_v, rel_hbm, bufr_v, semr, off), 0)
        dst = pl.ds(base + off, _CHUNK)
        pltpu.make_async_copy(ent_hbm.at[pl.ds(0, _CHUNK)], bufh_v, semh).wait()
        pltpu.sync_copy(bufh_v, hrows_hbm.at[dst])
        pltpu.make_async_copy(ent_hbm.at[pl.ds(0, _CHUNK)], buft_v, semt).wait()
        pltpu.sync_copy(buft_v, trows_hbm.at[dst])
        pltpu.make_async_copy(rel_hbm.at[pl.ds(0, _CHUNK)], bufr_v, semr).wait()
        pltpu.sync_copy(bufr_v, rrows_hbm.at[dst])


def _sc_gather(hidx, ridx, tidx, ent2, rel_emb):
    mesh = plsc.VectorSubcoreMesh(core_axis_name="c", subcore_axis_name="s")
    line_t = jax.ShapeDtypeStruct((BATCH, 2 * DIM), jnp.float32)
    rows_t = jax.ShapeDtypeStruct((BATCH, DIM), jnp.float32)
    k = functools.partial(
        pl.kernel,
        mesh=mesh,
        out_type=[line_t, rows_t, line_t],
        scratch_types=[
            pltpu.VMEM((_PER_W,), jnp.int32),
            pltpu.VMEM((_PER_W,), jnp.int32),
            pltpu.VMEM((_PER_W,), jnp.int32),
            pltpu.VMEM((_CHUNK, 2 * DIM), jnp.float32),
            pltpu.VMEM((_CHUNK, DIM), jnp.float32),
            pltpu.VMEM((_CHUNK, 2 * DIM), jnp.float32),
            pltpu.SemaphoreType.DMA,
            pltpu.SemaphoreType.DMA,
            pltpu.SemaphoreType.DMA,
        ],
    )(_gather_body)
    return k(hidx, ridx, tidx, ent2, rel_emb)


def _score_body(h2_ref, r_ref, t2_ref, mh_ref, mt_ref, o_ref):
    h2 = h2_ref[...]
    t2 = t2_ref[...]
    r = r_ref[...]
    h = jnp.where(mh_ref[...] > 0, h2[:, DIM:], h2[:, :DIM])
    t = jnp.where(mt_ref[...] > 0, t2[:, DIM:], t2[:, :DIM])
    sh = jnp.maximum(jnp.sum(jnp.abs(h), axis=1, keepdims=True), _EPS)
    st = jnp.maximum(jnp.sum(jnp.abs(t), axis=1, keepdims=True), _EPS)
    d = jnp.abs(h / sh + r - t / st)
    o_ref[...] = -jnp.sum(d, axis=1)


def _tc_score(h2, r, t2, mh, mt):
    blk = 4096
    grid = BATCH // blk
    lspec = pl.BlockSpec((blk, 2 * DIM), lambda i: (i, 0))
    rspec = pl.BlockSpec((blk, DIM), lambda i: (i, 0))
    mspec = pl.BlockSpec((blk, 1), lambda i: (i, 0))
    return pl.pallas_call(
        _score_body,
        grid=(grid,),
        in_specs=[lspec, rspec, lspec, mspec, mspec],
        out_specs=pl.BlockSpec((blk,), lambda i: (i,)),
        out_shape=jax.ShapeDtypeStruct((BATCH,), jnp.float32),
    )(h2, r, t2, mh, mt)


def kernel(head_idxs, rel_idxs, tail_idxs, ent_emb, rel_emb):
    hidx = head_idxs.astype(jnp.int32)
    ridx = rel_idxs.astype(jnp.int32)
    tidx = tail_idxs.astype(jnp.int32)
    ent2 = _tc_detile(ent_emb.T)
    mh = hidx >= SPLITP
    mt = tidx >= SPLITP
    h2idx = jnp.where(mh, hidx - SPLITP, hidx)
    t2idx = jnp.where(mt, tidx - SPLITP, tidx)
    h2, rrows, t2rows = _sc_gather(h2idx, ridx, t2idx, ent2, rel_emb)
    return _tc_score(h2, rrows, t2rows,
                     mh.astype(jnp.int32).reshape(BATCH, 1),
                     mt.astype(jnp.int32).reshape(BATCH, 1))


# speedup vs baseline: 1.0032x; 1.0032x over previous
import functools

import jax
import jax.numpy as jnp
from jax import lax
from jax.experimental import pallas as pl
from jax.experimental.pallas import tpu as pltpu
from jax.experimental.pallas import tpu_sc as plsc

DIM = 64
BATCH = 16384
NENT = 1000000

_KH = 8192
_NBLK = -(-NENT // (2 * _KH))
SPLITP = (_NBLK - 1) * _KH
_NROWS = _NBLK * _KH

_INFO = plsc.get_sparse_core_info()
_NC = _INFO.num_cores
_NS = _INFO.num_subcores
_NW = _NC * _NS
_PER_W = BATCH // _NW
_CHUNK = 256
_NCH = _PER_W // _CHUNK
_EPS = 1e-12


def _detile_body(x1_ref, x2_ref, o_ref):
    eye = jnp.eye(DIM, dtype=jnp.float32)
    dims = (((0,), (0,)), ((), ()))
    y1 = jax.lax.dot_general(x1_ref[...], eye, dims,
                             preferred_element_type=jnp.float32)
    y2 = jax.lax.dot_general(x2_ref[...], eye, dims,
                             preferred_element_type=jnp.float32)
    o_ref[...] = jnp.concatenate([y1, y2], axis=1)


def _tc_detile(entT):
    return pl.pallas_call(
        _detile_body,
        grid=(_NBLK,),
        in_specs=[pl.BlockSpec((DIM, _KH), lambda i: (0, i)),
                  pl.BlockSpec((DIM, _KH), lambda i: (0, i + _NBLK - 1))],
        out_specs=pl.BlockSpec((_KH, 2 * DIM), lambda i: (i, 0)),
        out_shape=jax.ShapeDtypeStruct((_NROWS, 2 * DIM), jnp.float32),
    )(entT, entT)


def _gather_body(hidx_hbm, ridx_hbm, tidx_hbm, ent_hbm, rel_hbm,
                 hrows_hbm, rrows_hbm, trows_hbm,
                 idxh_v, idxr_v, idxt_v, bufh_v, bufr_v, buft_v,
                 semh, semr, semt):
    wid = lax.axis_index("s") * _NC + lax.axis_index("c")
    base = wid * _PER_W

    pltpu.sync_copy(hidx_hbm.at[pl.ds(base, _PER_W)], idxh_v)
    pltpu.sync_copy(ridx_hbm.at[pl.ds(base, _PER_W)], idxr_v)
    pltpu.sync_copy(tidx_hbm.at[pl.ds(base, _PER_W)], idxt_v)

    def make_fire(idx_v, tab_hbm, buf_v, sem, off):
        def fire(g, carry):
            v = idx_v[pl.ds(off + g * 16, 16)]
            for i in range(16):
                pltpu.async_copy(tab_hbm.at[pl.ds(v[i], 1)],
                                 buf_v.at[pl.ds(g * 16 + i, 1)], sem)
            return carry
        return fire

    for j in range(_NCH):
        off = j * _CHUNK
        lax.fori_loop(0, _CHUNK // 16, make_fire(idxh_v, ent_hbm, bufh_v, semh, off), 0)
        lax.fori_loop(0, _CHUNK // 16, make_fire(idxt_v, ent_hbm, buft_v, semt, off), 0)
        lax.fori_loop(0, _CHUNK // 16, make_fire(idxr_v, rel_hbm, bufr_v, semr, off), 0)
        dst = pl.ds(base + off, _CHUNK)
        pltpu.make_async_copy(ent_hbm.at[pl.ds(0, _CHUNK)], bufh_v, semh).wait()
        pltpu.sync_copy(bufh_v, hrows_hbm.at[dst])
        pltpu.make_async_copy(ent_hbm.at[pl.ds(0, _CHUNK)], buft_v, semt).wait()
        pltpu.sync_copy(buft_v, trows_hbm.at[dst])
        pltpu.make_async_copy(rel_hbm.at[pl.ds(0, _CHUNK)], bufr_v, semr).wait()
        pltpu.sync_copy(bufr_v, rrows_hbm.at[dst])


def _sc_gather(hidx, ridx, tidx, ent2, rel_emb):
    mesh = plsc.VectorSubcoreMesh(core_axis_name="c", subcore_axis_name="s")
    line_t = jax.ShapeDtypeStruct((BATCH, 2 * DIM), jnp.float32)
    rows_t = jax.ShapeDtypeStruct((BATCH, DIM), jnp.float32)
    k = functools.partial(
        pl.kernel,
        mesh=mesh,
        out_type=[line_t, rows_t, line_t],
        scratch_types=[
            pltpu.VMEM((_PER_W,), jnp.int32),
            pltpu.VMEM((_PER_W,), jnp.int32),
            pltpu.VMEM((_PER_W,), jnp.int32),
            pltpu.VMEM((_CHUNK, 2 * DIM), jnp.float32),
            pltpu.VMEM((_CHUNK, DIM), jnp.float32),
            pltpu.VMEM((_CHUNK, 2 * DIM), jnp.float32),
            pltpu.SemaphoreType.DMA,
            pltpu.SemaphoreType.DMA,
            pltpu.SemaphoreType.DMA,
        ],
    )(_gather_body)
    return k(hidx, ridx, tidx, ent2, rel_emb)


def _score_body(h2_ref, r_ref, t2_ref, mh_ref, mt_ref, o_ref):
    h2 = h2_ref[...]
    t2 = t2_ref[...]
    r = r_ref[...]
    h = jnp.where(mh_ref[...] > 0, h2[:, DIM:], h2[:, :DIM])
    t = jnp.where(mt_ref[...] > 0, t2[:, DIM:], t2[:, :DIM])
    sh = jnp.maximum(jnp.sum(jnp.abs(h), axis=1, keepdims=True), _EPS)
    st = jnp.maximum(jnp.sum(jnp.abs(t), axis=1, keepdims=True), _EPS)
    d = jnp.abs(h / sh + r - t / st)
    o_ref[...] = -jnp.sum(d, axis=1)


def _tc_score(h2, r, t2, mh, mt):
    blk = 2048
    grid = BATCH // blk
    lspec = pl.BlockSpec((blk, 2 * DIM), lambda i: (i, 0))
    rspec = pl.BlockSpec((blk, DIM), lambda i: (i, 0))
    mspec = pl.BlockSpec((blk, 1), lambda i: (i, 0))
    return pl.pallas_call(
        _score_body,
        grid=(grid,),
        in_specs=[lspec, rspec, lspec, mspec, mspec],
        out_specs=pl.BlockSpec((blk,), lambda i: (i,)),
        out_shape=jax.ShapeDtypeStruct((BATCH,), jnp.float32),
    )(h2, r, t2, mh, mt)


def kernel(head_idxs, rel_idxs, tail_idxs, ent_emb, rel_emb):
    hidx = head_idxs.astype(jnp.int32)
    ridx = rel_idxs.astype(jnp.int32)
    tidx = tail_idxs.astype(jnp.int32)
    ent2 = _tc_detile(ent_emb.T)
    mh = hidx >= SPLITP
    mt = tidx >= SPLITP
    h2idx = jnp.where(mh, hidx - SPLITP, hidx)
    t2idx = jnp.where(mt, tidx - SPLITP, tidx)
    h2, rrows, t2rows = _sc_gather(h2idx, ridx, t2idx, ent2, rel_emb)
    return _tc_score(h2, rrows, t2rows,
                     mh.astype(jnp.int32).reshape(BATCH, 1),
                     mt.astype(jnp.int32).reshape(BATCH, 1))
